# baseline (speedup 1.0000x reference)
import functools

import jax
import jax.numpy as jnp
from jax import lax
from jax.experimental import pallas as pl
from jax.experimental.pallas import tpu as pltpu
from jax.experimental.pallas import tpu_sc as plsc

N_FIELDS = 26
VOCAB_P1 = 100001
EMBED_DIM = 64
BATCH = 4096

NUM_CORES = 2
NUM_SUBCORES = 16
NW = NUM_CORES * NUM_SUBCORES

CHUNK = BATCH // NW
BPW = N_FIELDS * CHUNK
ROWLEN = EMBED_DIM
BUFLEN = CHUNK * ROWLEN


@functools.partial(
    pl.kernel,
    out_type=jax.ShapeDtypeStruct((BATCH * N_FIELDS * EMBED_DIM,),
                                  jnp.float32),
    mesh=plsc.VectorSubcoreMesh(core_axis_name="c", subcore_axis_name="s"),
    scratch_types=[
        pltpu.VMEM((BPW,), jnp.int32),
        pltpu.VMEM_SHARED((NUM_SUBCORES, BPW), jnp.int32),
        pltpu.SMEM((2, CHUNK), jnp.int32),
        pltpu.VMEM((BUFLEN,), jnp.float32),
        pltpu.VMEM((BUFLEN,), jnp.float32),
        pltpu.SemaphoreType.DMA,
        pltpu.SemaphoreType.DMA,
        pltpu.SemaphoreType.DMA,
        pltpu.SemaphoreType.DMA,
        pltpu.SemaphoreType.DMA,
    ],
)
def _gather(tab_hbm, idx_hbm, out_hbm, idx_v, idx_sp, idx_s, buf0, buf1,
            semg0, semg1, semo0, semo1, sem_i):
    sid = lax.axis_index("s")
    wid = sid * NUM_CORES + lax.axis_index("c")
    base = wid * CHUNK

    bufs = (buf0, buf1)
    semg = (semg0, semg1)
    semo = (semo0, semo1)

    pltpu.sync_copy(idx_hbm.at[pl.ds(wid * BPW, BPW)], idx_v)
    pltpu.sync_copy(idx_v, idx_sp.at[sid])

    def fire(f, p):
        pltpu.async_copy(idx_sp.at[sid, pl.ds(f * CHUNK, CHUNK)],
                         idx_s.at[p], sem_i).wait()
        buf = bufs[p]
        fbase = f * VOCAB_P1 * EMBED_DIM

        def row(i):
            r = idx_s[p, i]
            pltpu.async_copy(
                tab_hbm.at[pl.ds(fbase + r * EMBED_DIM, EMBED_DIM)],
                buf.at[pl.ds(i * EMBED_DIM, EMBED_DIM)], semg[p])
        pl.loop(0, CHUNK)(row)

    def drain(sem_ref, p):
        pltpu.make_async_copy(
            tab_hbm.at[pl.ds(0, BUFLEN)], bufs[p], sem_ref).wait()

    def store(f, p):
        buf = bufs[p]
        obase = (base * N_FIELDS + f) * EMBED_DIM

        def row(i):
            pltpu.async_copy(
                buf.at[pl.ds(i * EMBED_DIM, EMBED_DIM)],
                out_hbm.at[pl.ds(obase + i * (N_FIELDS * EMBED_DIM),
                                 EMBED_DIM)], semo[p])
        pl.loop(0, CHUNK)(row)

    fire(0, 0)
    fire(1, 1)
    for f in range(2, N_FIELDS + 2):
        p = f % 2
        drain(semg[p], p)
        store(f - 2, p)
        drain(semo[p], p)
        if f < N_FIELDS:
            fire(f, p)


def kernel(x, tables):
    idx = (x.astype(jnp.int32)
           .reshape(NW, CHUNK, N_FIELDS)
           .transpose(0, 2, 1)
           .reshape(NW * BPW))
    tab_flat = tables.reshape(N_FIELDS * VOCAB_P1 * EMBED_DIM)
    out = _gather(tab_flat, idx)
    return out.reshape(BATCH, N_FIELDS, EMBED_DIM)


# speedup vs baseline: 8.3695x; 8.3695x over previous
import functools

import jax
import jax.numpy as jnp
from jax import lax
from jax.experimental import pallas as pl
from jax.experimental.pallas import tpu as pltpu
from jax.experimental.pallas import tpu_sc as plsc

N_FIELDS = 26
VOCAB_P1 = 100001
EMBED_DIM = 64
BATCH = 4096

NUM_CORES = 2
NUM_SUBCORES = 16
NW = NUM_CORES * NUM_SUBCORES

CHUNK = BATCH // NW
BPW = N_FIELDS * CHUNK


@functools.partial(
    pl.kernel,
    out_type=jax.ShapeDtypeStruct((N_FIELDS, BATCH, EMBED_DIM), jnp.float32),
    mesh=plsc.VectorSubcoreMesh(core_axis_name="c", subcore_axis_name="s"),
    scratch_types=[
        pltpu.VMEM((BPW,), jnp.int32),
        pltpu.VMEM_SHARED((NUM_SUBCORES, BPW), jnp.int32),
        pltpu.SMEM((2, CHUNK), jnp.int32),
        pltpu.VMEM((CHUNK, EMBED_DIM), jnp.float32),
        pltpu.VMEM((CHUNK, EMBED_DIM), jnp.float32),
        pltpu.SemaphoreType.DMA,
        pltpu.SemaphoreType.DMA,
        pltpu.SemaphoreType.DMA,
    ],
)
def _gather(tab_hbm, idx_hbm, out_hbm, idx_v, idx_sp, idx_s, buf0, buf1,
            sem0, sem1, sem_i):
    sid = lax.axis_index("s")
    wid = sid * NUM_CORES + lax.axis_index("c")
    base = wid * CHUNK

    bufs = (buf0, buf1)
    sems = (sem0, sem1)

    pltpu.sync_copy(idx_hbm.at[pl.ds(wid * BPW, BPW)], idx_v)
    pltpu.sync_copy(idx_v, idx_sp.at[sid])

    def fire(f, p):
        pltpu.async_copy(idx_sp.at[sid, pl.ds(f * CHUNK, CHUNK)],
                         idx_s.at[p], sem_i).wait()
        buf = bufs[p]

        def row(i):
            r = idx_s[p, i]
            pltpu.async_copy(tab_hbm.at[f].at[pl.ds(r, 1)],
                             buf.at[pl.ds(i, 1)], sems[p])
        pl.loop(0, CHUNK)(row)

    def drain_and_store(f, p):
        pltpu.make_async_copy(
            out_hbm.at[f].at[pl.ds(base, CHUNK)], bufs[p], sems[p]).wait()
        pltpu.sync_copy(bufs[p], out_hbm.at[f].at[pl.ds(base, CHUNK)])

    fire(0, 0)
    for f in range(1, N_FIELDS):
        fire(f, f % 2)
        drain_and_store(f - 1, (f - 1) % 2)
    drain_and_store(N_FIELDS - 1, (N_FIELDS - 1) % 2)


def kernel(x, tables):
    idx = (x.astype(jnp.int32)
           .reshape(NW, CHUNK, N_FIELDS)
           .transpose(0, 2, 1)
           .reshape(NW * BPW))
    out = _gather(tables, idx)
    return out.transpose(1, 0, 2)


# speedup vs baseline: 20.0404x; 2.3945x over previous
import functools

import jax
import jax.numpy as jnp
from jax import lax
from jax.experimental import pallas as pl
from jax.experimental.pallas import tpu as pltpu
from jax.experimental.pallas import tpu_sc as plsc

N_FIELDS = 26
VOCAB_P1 = 100001
EMBED_DIM = 64
BATCH = 4096

NUM_CORES = 2
NUM_SUBCORES = 16
NW = NUM_CORES * NUM_SUBCORES

ROWS = N_FIELDS * EMBED_DIM
RPW = ROWS // NW
NV = BATCH // 16


@functools.partial(
    pl.kernel,
    out_type=jax.ShapeDtypeStruct((ROWS, BATCH), jnp.float32),
    mesh=plsc.VectorSubcoreMesh(core_axis_name="c", subcore_axis_name="s"),
    scratch_types=[
        pltpu.VMEM((1, VOCAB_P1), jnp.float32),
        pltpu.VMEM((BATCH,), jnp.int32),
        pltpu.VMEM((1, BATCH), jnp.float32),
    ],
    compiler_params=pltpu.CompilerParams(needs_layout_passes=False),
)
def _gather(tab_hbm, idx_hbm, out_hbm, slab, idxv, orow):
    wid = lax.axis_index("s") * NUM_CORES + lax.axis_index("c")

    def do_row(j):
        g = wid + NW * j
        f = g // EMBED_DIM
        pltpu.sync_copy(tab_hbm.at[pl.ds(g, 1)], slab)
        pltpu.sync_copy(idx_hbm.at[pl.ds(f * BATCH, BATCH)], idxv)

        zeros = lax.iota(jnp.int32, 16) * 0

        def gath(v):
            rvec = idxv[pl.ds(v * 16, 16)]
            orow[0, pl.ds(v * 16, 16)] = plsc.load_gather(
                slab, [zeros, rvec])
        pl.loop(0, NV)(gath)

        pltpu.sync_copy(orow, out_hbm.at[pl.ds(g, 1)])

    pl.loop(0, RPW)(do_row)


def kernel(x, tables):
    tt2 = tables.transpose(0, 2, 1).reshape(ROWS, VOCAB_P1)
    idx = x.astype(jnp.int32).T.reshape(N_FIELDS * BATCH)
    out2 = _gather(tt2, idx)
    return out2.reshape(N_FIELDS, EMBED_DIM, BATCH).transpose(2, 0, 1)
